# baseline (speedup 1.0000x reference)
import jax
import jax.numpy as jnp
from jax import lax
from jax.experimental import pallas as pl
from jax.experimental.pallas import tpu as pltpu
from jax.experimental.pallas import tpu_sc as plsc

VOCAB = 100000
EMBED = 64
BATCH = 1024
CTX = 20
PAIR = 2 * EMBED

NC = 2
NS = 16
NW = NC * NS
B_PER_W = BATCH // NW

PACK_TV = 8192


def _pack_body(embT_ref, out_ref):
    blk = embT_ref[...]
    rows = blk.T
    ssq = jnp.sum(rows * rows, axis=-1, keepdims=True)
    norms = jnp.sqrt(ssq)
    scale = jnp.minimum(1.0, 1.0 / jnp.maximum(norms, 1e-7))
    rows = rows * (scale * jnp.float32(1.0 / CTX))
    out_ref[...] = jnp.concatenate(
        [rows, jnp.zeros((PACK_TV, PAIR - EMBED), jnp.float32)], axis=1)


def _pack(embT):
    return pl.pallas_call(
        _pack_body,
        grid=(pl.cdiv(VOCAB, PACK_TV),),
        in_specs=[pl.BlockSpec((EMBED, PACK_TV), lambda j: (0, j))],
        out_specs=pl.BlockSpec((PACK_TV, PAIR), lambda j: (j, 0)),
        out_shape=jax.ShapeDtypeStruct((VOCAB, PAIR), jnp.float32),
    )(embT)


def _sc_body(tab_hbm, idx_hbm, h_hbm, idx_v, rows_v, h_v, sem):
    wid = lax.axis_index("s") * NC + lax.axis_index("c")
    bbase = wid * B_PER_W
    lane0 = (wid // 4) * 128
    sub = (wid % 4) * B_PER_W
    pltpu.sync_copy(idx_hbm.at[pl.ds(0, CTX), pl.ds(lane0, 128)], idx_v)
    copies = []
    for t in range(CTX):
        copies.append(pltpu.async_copy(
            tab_hbm.at[idx_v.at[t, pl.ds(sub, B_PER_W)]],
            rows_v.at[t],
            sem))
    for cp in copies:
        cp.wait()

    def item_body(i, carry):
        accs = tuple(rows_v[0, i, pl.ds(c * 16, 16)] for c in range(8))

        def t_body(t, accs):
            return tuple(a + rows_v[t, i, pl.ds(c * 16, 16)]
                         for c, a in enumerate(accs))

        accs = lax.fori_loop(1, CTX, t_body, accs)
        for c in range(8):
            h_v[i, pl.ds(c * 16, 16)] = accs[c]
        return carry

    lax.fori_loop(0, B_PER_W, item_body, 0)
    pltpu.sync_copy(h_v, h_hbm.at[pl.ds(bbase, B_PER_W)])


def _sc_gather_pool(table, idxT):
    mesh = plsc.VectorSubcoreMesh(core_axis_name="c", subcore_axis_name="s")
    fn = pl.kernel(
        _sc_body,
        mesh=mesh,
        out_type=jax.ShapeDtypeStruct((BATCH, PAIR), jnp.float32),
        scratch_types=[
            pltpu.VMEM((CTX, 128), jnp.int32),
            pltpu.VMEM((CTX, B_PER_W, PAIR), jnp.float32),
            pltpu.VMEM((B_PER_W, PAIR), jnp.float32),
            pltpu.SemaphoreType.DMA,
        ],
    )
    return fn(table, idxT)


TV = 4096


def _tc_body(h_in_ref, w_ref, b_ref, out_ref, h_ref):
    @pl.when(pl.program_id(0) == 0)
    def _():
        h = h_in_ref[...][:, :EMBED]
        h_ref[...] = h.T.astype(jnp.bfloat16)

    outT = lax.dot_general(
        w_ref[...].astype(jnp.bfloat16), h_ref[...],
        dimension_numbers=(((0,), (0,)), ((), ())),
        preferred_element_type=jnp.float32)
    out_ref[...] = outT + b_ref[...][:, None]


def _project(h_in, WT, b):
    return pl.pallas_call(
        _tc_body,
        grid=(pl.cdiv(VOCAB, TV),),
        in_specs=[
            pl.BlockSpec((BATCH, PAIR), lambda j: (0, 0)),
            pl.BlockSpec((EMBED, TV), lambda j: (0, j)),
            pl.BlockSpec((TV,), lambda j: (j,)),
        ],
        out_specs=pl.BlockSpec((TV, BATCH), lambda j: (j, 0)),
        out_shape=jax.ShapeDtypeStruct((VOCAB, BATCH), jnp.float32),
        scratch_shapes=[pltpu.VMEM((EMBED, BATCH), jnp.bfloat16)],
    )(h_in, WT, b)


def kernel(x, emb, W, b):
    table = _pack(emb.T)
    h_in = _sc_gather_pool(table, x.T)
    outT = _project(h_in, W.T, b)
    return outT.T


# speedup vs baseline: 1.0789x; 1.0789x over previous
import jax
import jax.numpy as jnp
from jax import lax
from jax.experimental import pallas as pl
from jax.experimental.pallas import tpu as pltpu
from jax.experimental.pallas import tpu_sc as plsc

VOCAB = 100000
EMBED = 64
BATCH = 1024
CTX = 20
PAIR = 2 * EMBED

NC = 2
NS = 16
NW = NC * NS
B_PER_W = BATCH // NW

PACK_TV = 8192


def _pack_body(embT_ref, out_ref):
    blk = embT_ref[...]
    sq = blk * blk
    ssq = lax.dot_general(
        jnp.ones((1, EMBED), jnp.float32), sq,
        dimension_numbers=(((1,), (0,)), ((), ())),
        preferred_element_type=jnp.float32)
    norms = jnp.sqrt(ssq)
    scale = jnp.minimum(1.0, 1.0 / jnp.maximum(norms, 1e-7))
    rows = (blk * (scale * jnp.float32(1.0 / CTX))).T
    out_ref[...] = jnp.concatenate(
        [rows, jnp.zeros((PACK_TV, PAIR - EMBED), jnp.float32)], axis=1)


def _pack(embT):
    return pl.pallas_call(
        _pack_body,
        grid=(pl.cdiv(VOCAB, PACK_TV),),
        in_specs=[pl.BlockSpec((EMBED, PACK_TV), lambda j: (0, j))],
        out_specs=pl.BlockSpec((PACK_TV, PAIR), lambda j: (j, 0)),
        out_shape=jax.ShapeDtypeStruct((VOCAB, PAIR), jnp.float32),
    )(embT)


def _sc_body(tab_hbm, idx_hbm, h_hbm, idx_v, rows_v, h_v, sem):
    wid = lax.axis_index("s") * NC + lax.axis_index("c")
    bbase = wid * B_PER_W
    lane0 = (wid // 4) * 128
    sub = (wid % 4) * B_PER_W
    pltpu.sync_copy(idx_hbm.at[pl.ds(0, CTX), pl.ds(lane0, 128)], idx_v)
    copies = []
    for t in range(CTX):
        copies.append(pltpu.async_copy(
            tab_hbm.at[idx_v.at[t, pl.ds(sub, B_PER_W)]],
            rows_v.at[t],
            sem))
    for cp in copies:
        cp.wait()

    def item_body(i, carry):
        accs = tuple(rows_v[0, i, pl.ds(c * 16, 16)] for c in range(8))

        def t_body(t, accs):
            return tuple(a + rows_v[t, i, pl.ds(c * 16, 16)]
                         for c, a in enumerate(accs))

        accs = lax.fori_loop(1, CTX, t_body, accs)
        for c in range(8):
            h_v[i, pl.ds(c * 16, 16)] = accs[c]
        return carry

    lax.fori_loop(0, B_PER_W, item_body, 0)
    pltpu.sync_copy(h_v, h_hbm.at[pl.ds(bbase, B_PER_W)])


def _sc_gather_pool(table, idxT):
    mesh = plsc.VectorSubcoreMesh(core_axis_name="c", subcore_axis_name="s")
    fn = pl.kernel(
        _sc_body,
        mesh=mesh,
        out_type=jax.ShapeDtypeStruct((BATCH, PAIR), jnp.float32),
        scratch_types=[
            pltpu.VMEM((CTX, 128), jnp.int32),
            pltpu.VMEM((CTX, B_PER_W, PAIR), jnp.float32),
            pltpu.VMEM((B_PER_W, PAIR), jnp.float32),
            pltpu.SemaphoreType.DMA,
        ],
    )
    return fn(table, idxT)


TV = 4096


def _tc_body(h_in_ref, w_ref, b_ref, out_ref, h_ref):
    @pl.when(pl.program_id(0) == 0)
    def _():
        h = h_in_ref[...][:, :EMBED]
        h_ref[...] = h.T.astype(jnp.bfloat16)

    outT = lax.dot_general(
        w_ref[...].astype(jnp.bfloat16), h_ref[...],
        dimension_numbers=(((0,), (0,)), ((), ())),
        preferred_element_type=jnp.float32)
    out_ref[...] = outT + b_ref[...][:, None]


def _project(h_in, WT, b):
    return pl.pallas_call(
        _tc_body,
        grid=(pl.cdiv(VOCAB, TV),),
        in_specs=[
            pl.BlockSpec((BATCH, PAIR), lambda j: (0, 0)),
            pl.BlockSpec((EMBED, TV), lambda j: (0, j)),
            pl.BlockSpec((TV,), lambda j: (j,)),
        ],
        out_specs=pl.BlockSpec((TV, BATCH), lambda j: (j, 0)),
        out_shape=jax.ShapeDtypeStruct((VOCAB, BATCH), jnp.float32),
        scratch_shapes=[pltpu.VMEM((EMBED, BATCH), jnp.bfloat16)],
    )(h_in, WT, b)


def kernel(x, emb, W, b):
    table = _pack(emb.T)
    h_in = _sc_gather_pool(table, x.T)
    outT = _project(h_in, W.T, b)
    return outT.T
